# baseline (speedup 1.0000x reference)
import functools

import jax
import jax.numpy as jnp
from jax import lax
from jax.experimental import pallas as pl
from jax.experimental.pallas import tpu as pltpu
from jax.experimental.pallas import tpu_sc as plsc

H = 256
W = 256
C = 64
HW = H * W
L = 16
NW = 32
PPW = HW // NW
RPW = PPW // W
P = 128
NCH = PPW // P
NG = P // L
HALO_ROWS = 16

XOFF = (-1, 0, 1, -1, 0, 1, -1, 0, 1)
YOFF = (-1, -1, -1, 0, 0, 0, 1, 1, 1)

_CP = pltpu.CompilerParams(
    needs_layout_passes=False, use_tc_tiling_on_sc=False
)


def _reflect_scalar(t):
    t = jnp.where(t < 0, -t, t)
    return jnp.where(t > H - 1, 2 * (H - 1) - t, t)


def _score_and_select(inp_v, cand, idxb, outv, accb, g):
    rows_g = jnp.arange(L, dtype=jnp.int32) + g * L
    zero = jnp.zeros((L,), jnp.float32)
    for f in range(9):
        accb[f, :] = zero

    def cbody(c, carry):
        colv = jnp.full((L,), c, dtype=jnp.int32)
        inpv = plsc.load_gather(inp_v, [colv, rows_g])
        for f in range(9):
            plsc.addupdate(
                accb.at[f], inpv * plsc.load_gather(cand[f], [rows_g, colv])
            )
        return carry

    lax.fori_loop(0, C, cbody, 0)
    best_s = accb[0, :]
    best_i = idxb[0][pl.ds(g * L, L)]
    for f in range(1, 9):
        sf = accb[f, :]
        m = sf > best_s
        best_s = jnp.where(m, sf, best_s)
        best_i = jnp.where(m, idxb[f][pl.ds(g * L, L)], best_i)
    outv[pl.ds(g * L, L)] = best_i


def _gather_and_score(ref_hbm, oidx_hbm, inp_v, cand, idxb, outv, accb, sem,
                      off):
    handles = [
        pltpu.async_copy(ref_hbm.at[idxb[f]], cand[f], sem) for f in range(9)
    ]
    for h in handles:
        h.wait()

    def gbody(g, carry):
        _score_and_select(inp_v, cand, idxb, outv, accb, g)
        return carry

    lax.fori_loop(0, NG, gbody, 0)
    pltpu.sync_copy(outv, oidx_hbm.at[pl.ds(off, P)])


def _round_scratch():
    return (
        [pltpu.VMEM((C, P), jnp.float32)]
        + [pltpu.VMEM((P, C), jnp.float32) for _ in range(9)]
        + [pltpu.VMEM((P,), jnp.int32) for _ in range(9)]
        + [
            pltpu.VMEM((P,), jnp.int32),
            pltpu.VMEM((P,), jnp.int32),
            pltpu.VMEM((HALO_ROWS * W,), jnp.int32),
            pltpu.VMEM((9, L), jnp.float32),
            pltpu.SemaphoreType.DMA,
        ]
    )


@functools.lru_cache(maxsize=None)
def _make_prop(d):
    mesh = plsc.VectorSubcoreMesh(core_axis_name="c", subcore_axis_name="s")

    @functools.partial(
        pl.kernel,
        mesh=mesh,
        compiler_params=_CP,
        out_type=jax.ShapeDtypeStruct((HW,), jnp.int32),
        scratch_types=_round_scratch(),
    )
    def body(inp_hbm, ref_hbm, idx_hbm, oidx_hbm, inp_v, *rest):
        cand = rest[:9]
        idxb = rest[9:18]
        idxv, outv, halo, accb, sem = rest[18:]
        del idxv
        wid = lax.axis_index("s") * 2 + lax.axis_index("c")
        base = wid * PPW
        wrow0 = wid * RPW
        lo = jnp.clip(wrow0 - 4, 0, H - HALO_ROWS)
        pltpu.sync_copy(idx_hbm.at[pl.ds(lo * W, HALO_ROWS * W)], halo)

        def chunk(ch, carry):
            off = base + ch * P
            r = wrow0 + ch // (W // P)
            c0 = (ch % (W // P)) * P
            pltpu.sync_copy(inp_hbm.at[:, pl.ds(off, P)], inp_v)
            for f in range(9):
                yr = _reflect_scalar(r + YOFF[f] * d)
                rowbase = (yr - lo) * W
                for g in range(NG):
                    x = c0 + g * L + jnp.arange(L, dtype=jnp.int32)
                    xr = _reflect_scalar(x + XOFF[f] * d)
                    idxb[f][pl.ds(g * L, L)] = plsc.load_gather(
                        halo, [rowbase + xr]
                    )
            _gather_and_score(
                ref_hbm, oidx_hbm, inp_v, cand, idxb, outv, accb, sem, off
            )
            return carry

        lax.fori_loop(0, NCH, chunk, 0)

    return body


@functools.lru_cache(maxsize=None)
def _make_rand(k):
    mesh = plsc.VectorSubcoreMesh(core_axis_name="c", subcore_axis_name="s")

    @functools.partial(
        pl.kernel,
        mesh=mesh,
        compiler_params=_CP,
        out_type=jax.ShapeDtypeStruct((HW,), jnp.int32),
        scratch_types=_round_scratch(),
    )
    def body(inp_hbm, ref_hbm, idx_hbm, oidx_hbm, inp_v, *rest):
        cand = rest[:9]
        idxb = rest[9:18]
        idxv, outv, halo, accb, sem = rest[18:]
        del halo
        wid = lax.axis_index("s") * 2 + lax.axis_index("c")
        base = wid * PPW

        def chunk(ch, carry):
            off = base + ch * P
            pltpu.sync_copy(idx_hbm.at[pl.ds(off, P)], idxv)
            pltpu.sync_copy(inp_hbm.at[:, pl.ds(off, P)], inp_v)
            for g in range(NG):
                v = idxv[pl.ds(g * L, L)]
                x = v & (W - 1)
                y = v >> 8
                for f in range(9):
                    cx = (x + (k * XOFF[f] + W)) & (W - 1)
                    cy = (y + (k * YOFF[f] + H)) & (H - 1)
                    idxb[f][pl.ds(g * L, L)] = (cy << 8) | cx
            _gather_and_score(
                ref_hbm, oidx_hbm, inp_v, cand, idxb, outv, accb, sem, off
            )
            return carry

        lax.fori_loop(0, NCH, chunk, 0)

    return body


@functools.lru_cache(maxsize=None)
def _make_final():
    mesh = plsc.VectorSubcoreMesh(core_axis_name="c", subcore_axis_name="s")

    @functools.partial(
        pl.kernel,
        mesh=mesh,
        compiler_params=_CP,
        out_type=jax.ShapeDtypeStruct((HW,), jnp.float32),
        scratch_types=[
            pltpu.VMEM((C, P), jnp.float32),
            pltpu.VMEM((P, C), jnp.float32),
            pltpu.VMEM((P,), jnp.int32),
            pltpu.VMEM((P,), jnp.float32),
            pltpu.VMEM((1, L), jnp.float32),
            pltpu.SemaphoreType.DMA,
        ],
    )
    def body(inp_hbm, ref_hbm, idx_hbm, s_hbm, inp_v, cand, idxv, sv, accb,
             sem):
        wid = lax.axis_index("s") * 2 + lax.axis_index("c")
        base = wid * PPW

        def chunk(ch, carry):
            off = base + ch * P
            pltpu.sync_copy(idx_hbm.at[pl.ds(off, P)], idxv)
            pltpu.sync_copy(inp_hbm.at[:, pl.ds(off, P)], inp_v)
            pltpu.async_copy(ref_hbm.at[idxv], cand, sem).wait()

            def gbody(g, carry):
                rows_g = jnp.arange(L, dtype=jnp.int32) + g * L
                accb[0, :] = jnp.zeros((L,), jnp.float32)

                def cbody(c, inner):
                    colv = jnp.full((L,), c, dtype=jnp.int32)
                    inpv = plsc.load_gather(inp_v, [colv, rows_g])
                    plsc.addupdate(
                        accb.at[0],
                        inpv * plsc.load_gather(cand, [rows_g, colv]),
                    )
                    return inner

                lax.fori_loop(0, C, cbody, 0)
                sv[pl.ds(g * L, L)] = accb[0, :]
                return carry

            lax.fori_loop(0, NG, gbody, 0)
            pltpu.sync_copy(sv, s_hbm.at[pl.ds(off, P)])
            return carry

        lax.fori_loop(0, NCH, chunk, 0)

    return body


def kernel(input_map, ref_map, inref_x, inref_y, is_final, iteration_count,
           input_minWH, ref_minWH):
    del is_final, iteration_count, input_minWH, ref_minWH
    inp2d = input_map.reshape(C, HW)
    ref_t = jnp.transpose(ref_map.reshape(C, HW))
    ix = inref_x.astype(jnp.int32).reshape(-1)
    iy = inref_y.astype(jnp.int32).reshape(-1)
    idx = iy * W + ix
    for d in (1, 2, 4):
        idx = _make_prop(d)(inp2d, ref_t, idx)
    for k in (1, 2, 4):
        idx = _make_rand(k)(inp2d, ref_t, idx)
    s = _make_final()(inp2d, ref_t, idx)
    return idx.reshape(H, W), s.reshape(1, 1, H, W)


# speedup vs baseline: 3.7311x; 3.7311x over previous
import functools

import jax
import jax.numpy as jnp
from jax import lax
from jax.experimental import pallas as pl
from jax.experimental.pallas import tpu as pltpu
from jax.experimental.pallas import tpu_sc as plsc

H = 256
W = 256
C = 64
HW = H * W
L = 16
NW = 32
PPW = HW // NW
RPW = PPW // W
P = 128
NCH = PPW // P
NG = P // L
CP = C + 1
HALO_ROWS = 16

XOFF = (-1, 0, 1, -1, 0, 1, -1, 0, 1)
YOFF = (-1, -1, -1, 0, 0, 0, 1, 1, 1)

_CP = pltpu.CompilerParams(
    needs_layout_passes=False, use_tc_tiling_on_sc=False
)


def _reflect_scalar(t):
    t = jnp.where(t < 0, -t, t)
    return jnp.where(t > H - 1, 2 * (H - 1) - t, t)


def _score_and_select(inp_v, cand, idxb, outv, accb, g):
    del accb
    rows_g = jnp.arange(L, dtype=jnp.int32) + g * L

    def cbody(c, accs):
        colv = jnp.full((L,), c, dtype=jnp.int32)
        inpv = plsc.load_gather(inp_v, [colv, rows_g])
        return tuple(
            accs[f] + inpv * plsc.load_gather(cand[f], [rows_g, colv])
            for f in range(9)
        )

    accs = lax.fori_loop(
        0, C, cbody, tuple(jnp.zeros((L,), jnp.float32) for _ in range(9))
    )
    best_s = accs[0]
    best_i = idxb[0][pl.ds(g * L, L)]
    for f in range(1, 9):
        m = accs[f] > best_s
        best_s = jnp.where(m, accs[f], best_s)
        best_i = jnp.where(m, idxb[f][pl.ds(g * L, L)], best_i)
    outv[pl.ds(g * L, L)] = best_i


def _gather_and_score(ref_hbm, oidx_hbm, inp_v, cand, idxb, outv, accb, sem,
                      off):
    handles = [
        pltpu.async_copy(ref_hbm.at[idxb[f]], cand[f], sem) for f in range(9)
    ]
    for h in handles:
        h.wait()

    def gbody(g, carry):
        _score_and_select(inp_v, cand, idxb, outv, accb, g)
        return carry

    lax.fori_loop(0, NG, gbody, 0)
    pltpu.sync_copy(outv, oidx_hbm.at[pl.ds(off, P)])


def _round_scratch():
    return (
        [pltpu.VMEM((C, P), jnp.float32)]
        + [pltpu.VMEM((P, CP), jnp.float32) for _ in range(9)]
        + [pltpu.VMEM((P,), jnp.int32) for _ in range(9)]
        + [
            pltpu.VMEM((P,), jnp.int32),
            pltpu.VMEM((P,), jnp.int32),
            pltpu.VMEM((HALO_ROWS * W,), jnp.int32),
            pltpu.SemaphoreType.DMA,
        ]
    )


@functools.lru_cache(maxsize=None)
def _make_prop(d):
    mesh = plsc.VectorSubcoreMesh(core_axis_name="c", subcore_axis_name="s")

    @functools.partial(
        pl.kernel,
        mesh=mesh,
        compiler_params=_CP,
        out_type=jax.ShapeDtypeStruct((HW,), jnp.int32),
        scratch_types=_round_scratch(),
    )
    def body(inp_hbm, ref_hbm, idx_hbm, oidx_hbm, inp_v, *rest):
        cand = rest[:9]
        idxb = rest[9:18]
        idxv, outv, halo, sem = rest[18:]
        del idxv
        accb = None
        wid = lax.axis_index("s") * 2 + lax.axis_index("c")
        base = wid * PPW
        wrow0 = wid * RPW
        lo = jnp.clip(wrow0 - 4, 0, H - HALO_ROWS)
        pltpu.sync_copy(idx_hbm.at[pl.ds(lo * W, HALO_ROWS * W)], halo)

        def chunk(ch, carry):
            off = base + ch * P
            r = wrow0 + ch // (W // P)
            c0 = (ch % (W // P)) * P
            pltpu.sync_copy(inp_hbm.at[:, pl.ds(off, P)], inp_v)
            for f in range(9):
                yr = _reflect_scalar(r + YOFF[f] * d)
                rowbase = (yr - lo) * W
                for g in range(NG):
                    x = c0 + g * L + jnp.arange(L, dtype=jnp.int32)
                    xr = _reflect_scalar(x + XOFF[f] * d)
                    idxb[f][pl.ds(g * L, L)] = plsc.load_gather(
                        halo, [rowbase + xr]
                    )
            _gather_and_score(
                ref_hbm, oidx_hbm, inp_v, cand, idxb, outv, accb, sem, off
            )
            return carry

        lax.fori_loop(0, NCH, chunk, 0)

    return body


@functools.lru_cache(maxsize=None)
def _make_rand(k):
    mesh = plsc.VectorSubcoreMesh(core_axis_name="c", subcore_axis_name="s")

    @functools.partial(
        pl.kernel,
        mesh=mesh,
        compiler_params=_CP,
        out_type=jax.ShapeDtypeStruct((HW,), jnp.int32),
        scratch_types=_round_scratch(),
    )
    def body(inp_hbm, ref_hbm, idx_hbm, oidx_hbm, inp_v, *rest):
        cand = rest[:9]
        idxb = rest[9:18]
        idxv, outv, halo, sem = rest[18:]
        del halo
        accb = None
        wid = lax.axis_index("s") * 2 + lax.axis_index("c")
        base = wid * PPW

        def chunk(ch, carry):
            off = base + ch * P
            pltpu.sync_copy(idx_hbm.at[pl.ds(off, P)], idxv)
            pltpu.sync_copy(inp_hbm.at[:, pl.ds(off, P)], inp_v)
            for g in range(NG):
                v = idxv[pl.ds(g * L, L)]
                x = v & (W - 1)
                y = v >> 8
                for f in range(9):
                    cx = (x + (k * XOFF[f] + W)) & (W - 1)
                    cy = (y + (k * YOFF[f] + H)) & (H - 1)
                    idxb[f][pl.ds(g * L, L)] = (cy << 8) | cx
            _gather_and_score(
                ref_hbm, oidx_hbm, inp_v, cand, idxb, outv, accb, sem, off
            )
            return carry

        lax.fori_loop(0, NCH, chunk, 0)

    return body


@functools.lru_cache(maxsize=None)
def _make_final():
    mesh = plsc.VectorSubcoreMesh(core_axis_name="c", subcore_axis_name="s")

    @functools.partial(
        pl.kernel,
        mesh=mesh,
        compiler_params=_CP,
        out_type=jax.ShapeDtypeStruct((HW,), jnp.float32),
        scratch_types=[
            pltpu.VMEM((C, P), jnp.float32),
            pltpu.VMEM((P, CP), jnp.float32),
            pltpu.VMEM((P,), jnp.int32),
            pltpu.VMEM((P,), jnp.float32),
            pltpu.SemaphoreType.DMA,
        ],
    )
    def body(inp_hbm, ref_hbm, idx_hbm, s_hbm, inp_v, cand, idxv, sv, sem):
        wid = lax.axis_index("s") * 2 + lax.axis_index("c")
        base = wid * PPW

        def chunk(ch, carry):
            off = base + ch * P
            pltpu.sync_copy(idx_hbm.at[pl.ds(off, P)], idxv)
            pltpu.sync_copy(inp_hbm.at[:, pl.ds(off, P)], inp_v)
            pltpu.async_copy(ref_hbm.at[idxv], cand, sem).wait()

            def gbody(g, carry):
                rows_g = jnp.arange(L, dtype=jnp.int32) + g * L

                def cbody(c, acc):
                    colv = jnp.full((L,), c, dtype=jnp.int32)
                    inpv = plsc.load_gather(inp_v, [colv, rows_g])
                    return acc + inpv * plsc.load_gather(cand, [rows_g, colv])

                acc = lax.fori_loop(0, C, cbody, jnp.zeros((L,), jnp.float32))
                sv[pl.ds(g * L, L)] = acc
                return carry

            lax.fori_loop(0, NG, gbody, 0)
            pltpu.sync_copy(sv, s_hbm.at[pl.ds(off, P)])
            return carry

        lax.fori_loop(0, NCH, chunk, 0)

    return body


def kernel(input_map, ref_map, inref_x, inref_y, is_final, iteration_count,
           input_minWH, ref_minWH):
    del is_final, iteration_count, input_minWH, ref_minWH
    inp2d = input_map.reshape(C, HW)
    ref_t = jnp.pad(
        jnp.transpose(ref_map.reshape(C, HW)), ((0, 0), (0, CP - C))
    )
    ix = inref_x.astype(jnp.int32).reshape(-1)
    iy = inref_y.astype(jnp.int32).reshape(-1)
    idx = iy * W + ix
    for d in (1, 2, 4):
        idx = _make_prop(d)(inp2d, ref_t, idx)
    for k in (1, 2, 4):
        idx = _make_rand(k)(inp2d, ref_t, idx)
    s = _make_final()(inp2d, ref_t, idx)
    return idx.reshape(H, W), s.reshape(1, 1, H, W)
